# baseline (speedup 1.0000x reference)
import functools

import jax
import jax.numpy as jnp
from jax import lax
from jax.experimental import pallas as pl
from jax.experimental.pallas import tpu as pltpu
from jax.experimental.pallas import tpu_sc as plsc

_NC = 2
_NS = 16
_CHUNK = 16


def _sc_copy(tokens, d_model, x_hbm, o_hbm, buf):
    rows_per_worker = tokens // (_NC * _NS)
    wid = lax.axis_index("s") * _NC + lax.axis_index("c")
    base = wid * rows_per_worker

    @pl.loop(0, rows_per_worker // _CHUNK)
    def _(j):
        off = base + j * _CHUNK
        pltpu.sync_copy(x_hbm.at[pl.ds(off, _CHUNK), :], buf)
        pltpu.sync_copy(buf, o_hbm.at[pl.ds(off, _CHUNK), :])


def kernel(hidden_states, routing_weights, selected_experts):
    del routing_weights, selected_experts
    tokens, d_model = hidden_states.shape
    mesh = plsc.VectorSubcoreMesh(core_axis_name="c", subcore_axis_name="s")
    f = pl.kernel(
        functools.partial(_sc_copy, tokens, d_model),
        out_type=jax.ShapeDtypeStruct((tokens, d_model), hidden_states.dtype),
        mesh=mesh,
        scratch_types=[pltpu.VMEM((_CHUNK, d_model), jnp.float32)],
    )
    return f(hidden_states)


# speedup vs baseline: 1.3369x; 1.3369x over previous
import jax
import jax.numpy as jnp
from jax.experimental import pallas as pl
from jax.experimental.pallas import tpu as pltpu


def _copy_block(x_ref, o_ref):
    o_ref[...] = x_ref[...]


def kernel(hidden_states, routing_weights, selected_experts):
    del routing_weights, selected_experts
    tokens, d_model = hidden_states.shape
    block_rows = 1016
    return pl.pallas_call(
        _copy_block,
        grid=(pl.cdiv(tokens, block_rows),),
        in_specs=[pl.BlockSpec((block_rows, d_model), lambda i: (i, 0))],
        out_specs=pl.BlockSpec((block_rows, d_model), lambda i: (i, 0)),
        out_shape=jax.ShapeDtypeStruct((tokens, d_model), hidden_states.dtype),
        compiler_params=pltpu.CompilerParams(dimension_semantics=("arbitrary",), vmem_limit_bytes=134217728),
    )(hidden_states)
